# baseline (speedup 1.0000x reference)
import jax
import jax.numpy as jnp
from jax import lax
from jax.experimental import pallas as pl
from jax.experimental.pallas import tpu as pltpu
from jax.experimental.pallas import tpu_sc as plsc

B, H, W, NSRC = 8, 2048, 2048, 128
STRIDE = H // NSRC

NC, NS, L = 2, 16, 16
NW = NC * NS
RPWR = (B * H) // NW
CH = 16
NCH = RPWR // CH


def _sc_body(yf, ycol, xamp, out, bufA, bufB, yv, xv, siA, siB, soA, soB):
    c_ax = lax.axis_index("c")
    s_ax = lax.axis_index("s")
    w = s_ax * NC + c_ax
    base = w * RPWR
    pltpu.sync_copy(ycol, yv)
    pltpu.sync_copy(xamp, xv)
    iot = lax.iota(jnp.int32, L)
    zer = iot * 0
    m0 = iot == 0
    bufs = (bufA, bufB)
    isems = (siA, siB)
    osems = (soA, soB)
    in_cp = [None] * NCH
    out_cp = [None] * NCH
    in_cp[0] = pltpu.async_copy(yf.at[pl.ds(base, CH)], bufs[0], isems[0])
    for c in range(NCH):
        buf = bufs[c % 2]
        in_cp[c].wait()
        r = base + c * CH
        i = (r % H) // STRIDE
        ivec = zer + i
        yk = plsc.load_gather(yv, [ivec])
        xk = plsc.load_gather(xv, [ivec])
        vals = plsc.load_gather(buf, [zer, yk], mask=m0)
        plsc.store_scatter(buf, [zer, yk], vals + xk, mask=m0)
        out_cp[c] = pltpu.async_copy(buf, out.at[pl.ds(r, CH)], osems[c % 2])
        if c + 1 < NCH:
            if c >= 1:
                out_cp[c - 1].wait()
            in_cp[c + 1] = pltpu.async_copy(
                yf.at[pl.ds(r + CH, CH)], bufs[(c + 1) % 2], isems[(c + 1) % 2])
    out_cp[NCH - 1].wait()
    out_cp[NCH - 2].wait()


def _fast(Y, X, x, y):
    Yf = Y.reshape(B * H, W)
    mesh = plsc.VectorSubcoreMesh(core_axis_name="c", subcore_axis_name="s")
    out = pl.kernel(
        _sc_body,
        out_type=jax.ShapeDtypeStruct((B * H, W), jnp.float32),
        mesh=mesh,
        scratch_types=[
            pltpu.VMEM((CH, W), jnp.float32),
            pltpu.VMEM((CH, W), jnp.float32),
            pltpu.VMEM((NSRC,), jnp.int32),
            pltpu.VMEM((NSRC,), jnp.float32),
            pltpu.SemaphoreType.DMA,
            pltpu.SemaphoreType.DMA,
            pltpu.SemaphoreType.DMA,
            pltpu.SemaphoreType.DMA,
        ],
        compiler_params=pltpu.CompilerParams(needs_layout_passes=False),
    )(Yf, y, X)
    return out.reshape(B, H, W)


FR = 1024
NBLK = (B * H) // FR


def _gen_body(lo_ref, hi_ref, xf_ref, yf_ref, xvf_ref, yin, yout):
    g = pl.program_id(0)
    yout[...] = yin[...]
    r0 = g * FR

    def upd(i, carry):
        dr = xf_ref[i] - r0
        yi = yf_ref[i]
        xv = xvf_ref[i]
        col = lax.broadcasted_iota(jnp.int32, (1, W), 1)
        row = yout[pl.ds(dr, 1), :]
        yout[pl.ds(dr, 1), :] = row + jnp.where(col == yi, xv, 0.0)
        return carry

    lax.fori_loop(lo_ref[g], hi_ref[g], upd, 0)


def _generic(Y, X, x, y):
    Yf = Y.reshape(B * H, W)
    xf = (jnp.arange(B, dtype=jnp.int32)[:, None] * H + x[None, :]).reshape(-1)
    yf = jnp.broadcast_to(y, (B, NSRC)).reshape(-1)
    xvf = jnp.broadcast_to(X, (B, NSRC)).reshape(-1)

    block_starts = jnp.arange(NBLK, dtype=jnp.int32) * FR
    lo = jnp.searchsorted(xf, block_starts, side="left").astype(jnp.int32)
    hi = jnp.searchsorted(xf, block_starts + FR, side="left").astype(jnp.int32)

    grid_spec = pltpu.PrefetchScalarGridSpec(
        num_scalar_prefetch=5,
        grid=(NBLK,),
        in_specs=[pl.BlockSpec((FR, W), lambda g, *refs: (g, 0))],
        out_specs=pl.BlockSpec((FR, W), lambda g, *refs: (g, 0)),
    )
    out = pl.pallas_call(
        _gen_body,
        grid_spec=grid_spec,
        out_shape=jax.ShapeDtypeStruct((B * H, W), jnp.float32),
    )(lo, hi, xf, yf, xvf, Yf)
    return out.reshape(B, H, W)


def kernel(Y, X, x, y):
    structured = jnp.all(x == jnp.arange(NSRC, dtype=jnp.int32) * STRIDE)
    return lax.cond(structured, _fast, _generic, Y, X, x, y)
